# baseline (speedup 1.0000x reference)
import functools

import jax
import jax.numpy as jnp
from jax import lax
from jax.experimental import pallas as pl
from jax.experimental.pallas import tpu as pltpu
from jax.experimental.pallas import tpu_sc as plsc

_B = 4096
_D = 512
_C = 1000
_L = 16
_NCORE = 2
_NSUB = 16
_NW = _NCORE * _NSUB
_SPW = _B // _NSUB
_ROWS = 32
_NCH = _SPW // _ROWS
_CT = 1024
_CPW = _CT // _NSUB
_FV = _D // _L


_GATHER_DNUMS = lax.GatherDimensionNumbers(
    offset_dims=(), collapsed_slice_dims=(0,), start_index_map=(0,))


def _bcast_lane(vec, k):
    idx = jnp.full((_L, 1), k, jnp.int32)
    return lax.gather(vec, idx, _GATHER_DNUMS, slice_sizes=(1,),
                      mode=lax.GatherScatterMode.PROMISE_IN_BOUNDS)


def _sc_body(m1_hbm, m2_hbm, tgt_hbm, cent_hbm, out_hbm,
             tgt_v, idx_v, buf_v, cent_v, cnt_v, inv_v, res_v, sums_sp,
             tsem, csem, msem, ssem):
    cid = lax.axis_index("c")
    sid = lax.axis_index("s")
    wid = sid * _NCORE + cid
    s0 = sid * _SPW

    zeros = jnp.zeros((_L,), jnp.float32)

    def _inbound(c):
        p = c % 2
        sl = pl.ds(s0 + c * _ROWS, _ROWS)
        a = pltpu.make_async_copy(m1_hbm.at[sl], buf_v.at[p], msem.at[p])
        b = pltpu.make_async_copy(m2_hbm.at[sl], buf_v.at[p], msem.at[p])
        return a, b

    def _start_inbound(c):
        a, b = _inbound(c)

        @pl.when(cid == 0)
        def _():
            a.start()

        @pl.when(cid == 1)
        def _():
            b.start()

    def _wait_inbound(c):
        a, _unused = _inbound(c)
        a.wait()

    def _scatter(c):
        p = c % 2
        return pltpu.make_async_copy(
            buf_v.at[p], sums_sp.at[idx_v.at[c]], ssem.at[p])

    c0 = sid * _CPW
    cs = jnp.minimum(c0, _C - _CPW)
    tgt_cp = pltpu.make_async_copy(tgt_hbm, tgt_v, tsem)
    tgt_cp.start()
    cent_cp = pltpu.make_async_copy(
        cent_hbm.at[pl.ds(cs * _D, _CPW * _D)], cent_v, csem)
    cent_cp.start()
    _start_inbound(0)

    @plsc.parallel_loop(0, _CT // _L, unroll=4)
    def _zero_cnt(i):
        cnt_v[pl.ds(i * _L, _L)] = zeros

    @plsc.parallel_loop(0, _ROWS * _FV, unroll=4)
    def _zero_buf(i):
        buf_v[1, i // _FV, pl.ds((i % _FV) * _L, _L)] = zeros

    pltpu.sync_copy(buf_v.at[1], sums_sp.at[pl.ds(sid * _CPW, _ROWS)])
    pltpu.sync_copy(buf_v.at[1], sums_sp.at[pl.ds(sid * _CPW + _ROWS, _ROWS)])

    tgt_cp.wait()

    @plsc.parallel_loop(0, _B // _L, unroll=2)
    def _prep(g):
        tvec = tgt_v[pl.ds(g * _L, _L)]
        dup, last = plsc.scan_count(tvec)
        plsc.addupdate_scatter(
            cnt_v, [tvec], dup.astype(jnp.float32), mask=last)

    @plsc.parallel_loop(0, _SPW // _L, unroll=2)
    def _prep_idx(g):
        c = g // (_ROWS // _L)
        off = (g % (_ROWS // _L)) * _L
        idx_v[c, pl.ds(off, _L)] = tgt_v[pl.ds(s0 + g * _L, _L)]

    plsc.subcore_barrier()

    for c in range(_NCH):
        _wait_inbound(c)
        _scatter(c).start(add=True)
        if c >= 1:
            _scatter(c - 1).wait()
        if c + 1 < _NCH:
            _start_inbound(c + 1)
    _scatter(_NCH - 1).wait()

    plsc.subcore_barrier()

    pltpu.sync_copy(sums_sp.at[pl.ds(cs, _ROWS)], buf_v.at[0])
    pltpu.sync_copy(sums_sp.at[pl.ds(cs + _ROWS, _ROWS)], buf_v.at[1])

    @plsc.parallel_loop(0, _CT // _L, unroll=4)
    def _inv_cnt(i):
        inv_v[pl.ds(i * _L, _L)] = 1.0 / jnp.maximum(cnt_v[pl.ds(i * _L, _L)], 1.0)

    cent_cp.wait()

    accs0 = (zeros, zeros, zeros, zeros)

    @plsc.parallel_loop(0, _CPW, carry=accs0)
    def _class_loop(ci, accs):
        cw = cnt_v[pl.ds(cs + ci, _L)]
        iw = inv_v[pl.ds(cs + ci, _L)]
        cb = _bcast_lane(cw, 0)
        inv = _bcast_lane(iw, 0)
        keep = jnp.full((_L,), 0, jnp.int32) + (cs + ci - c0) >= 0
        cb = jnp.where(keep, cb, 0.0)
        accs = list(accs)
        for f in range(_FV):
            d = buf_v[ci // _ROWS, ci % _ROWS, pl.ds(f * _L, _L)] * inv \
                - cent_v[pl.ds(ci * _D + f * _L, _L)]
            a = jnp.abs(d)
            h = jnp.where(a < 1.0, 0.5 * d * d, a - 0.5)
            accs[f % 4] = accs[f % 4] + cb * h
        return tuple(accs)

    accs = _class_loop
    res_v[...] = (accs[0] + accs[1]) + (accs[2] + accs[3])
    pltpu.sync_copy(res_v, out_hbm.at[wid])


_sc_kernel = functools.partial(
    pl.kernel,
    out_type=jax.ShapeDtypeStruct((_NW, _L), jnp.float32),
    mesh=plsc.VectorSubcoreMesh(core_axis_name="c", subcore_axis_name="s"),
    compiler_params=pltpu.CompilerParams(
        use_tc_tiling_on_sc=False, needs_layout_passes=False),
    scratch_types=[
        pltpu.VMEM((_B,), jnp.int32),
        pltpu.VMEM((_NCH, _ROWS), jnp.int32),
        pltpu.VMEM((2, _ROWS, _D), jnp.float32),
        pltpu.VMEM((_CPW * _D,), jnp.float32),
        pltpu.VMEM((_CT,), jnp.float32),
        pltpu.VMEM((_CT,), jnp.float32),
        pltpu.VMEM((_L,), jnp.float32),
        pltpu.VMEM_SHARED((_CT, _D), jnp.float32),
        pltpu.SemaphoreType.DMA,
        pltpu.SemaphoreType.DMA,
        pltpu.SemaphoreType.DMA((2,)),
        pltpu.SemaphoreType.DMA((2,)),
    ],
)(_sc_body)


def _tc_reduce_body(x_ref, o_ref):
    o_ref[...] = jnp.sum(x_ref[...]).reshape(1, 1) * (1.0 / (_B * _D))


def kernel(modal1_inputs, modal2_inputs, targets, centers_param):
    partials = _sc_kernel(modal1_inputs, modal2_inputs, targets,
                          centers_param.reshape(-1))
    out = pl.pallas_call(
        _tc_reduce_body,
        out_shape=jax.ShapeDtypeStruct((1, 1), jnp.float32),
    )(partials)
    return out[0, 0]


# speedup vs baseline: 1.0411x; 1.0411x over previous
import functools

import jax
import jax.numpy as jnp
from jax import lax
from jax.experimental import pallas as pl
from jax.experimental.pallas import tpu as pltpu
from jax.experimental.pallas import tpu_sc as plsc

_B = 4096
_D = 512
_C = 1000
_L = 16
_NCORE = 2
_NSUB = 16
_NW = _NCORE * _NSUB
_SPW = _B // _NSUB
_ROWS = 64
_NCH = _SPW // _ROWS
_CT = 1024
_CPW = _CT // _NSUB
_FV = _D // _L


_GATHER_DNUMS = lax.GatherDimensionNumbers(
    offset_dims=(), collapsed_slice_dims=(0,), start_index_map=(0,))


def _bcast_lane(vec, k):
    idx = jnp.full((_L, 1), k, jnp.int32)
    return lax.gather(vec, idx, _GATHER_DNUMS, slice_sizes=(1,),
                      mode=lax.GatherScatterMode.PROMISE_IN_BOUNDS)


def _sc_body(m1_hbm, m2_hbm, tgt_hbm, cent_hbm, out_hbm,
             tgt_v, idx_v, buf_v, cnt_v, inv_v, res_v, sums_sp,
             tsem, csem, msem, ssem):
    cid = lax.axis_index("c")
    sid = lax.axis_index("s")
    wid = sid * _NCORE + cid
    s0 = sid * _SPW

    zeros = jnp.zeros((_L,), jnp.float32)

    def _inbound(c):
        p = c % 2
        sl = pl.ds(s0 + c * _ROWS, _ROWS)
        a = pltpu.make_async_copy(m1_hbm.at[sl], buf_v.at[p], msem.at[p])
        b = pltpu.make_async_copy(m2_hbm.at[sl], buf_v.at[p], msem.at[p])
        return a, b

    def _start_inbound(c):
        a, b = _inbound(c)

        @pl.when(cid == 0)
        def _():
            a.start()

        @pl.when(cid == 1)
        def _():
            b.start()

    def _wait_inbound(c):
        a, _unused = _inbound(c)
        a.wait()

    def _scatter(c):
        p = c % 2
        return pltpu.make_async_copy(
            buf_v.at[p], sums_sp.at[idx_v.at[c]], ssem.at[p])

    tgt_cp = pltpu.make_async_copy(tgt_hbm, tgt_v, tsem)
    tgt_cp.start()
    _start_inbound(0)

    @plsc.parallel_loop(0, _CT // _L, unroll=4)
    def _zero_cnt(i):
        cnt_v[pl.ds(i * _L, _L)] = zeros

    @plsc.parallel_loop(0, _ROWS * _FV, unroll=4)
    def _zero_buf(i):
        buf_v[1, i // _FV, pl.ds((i % _FV) * _L, _L)] = zeros

    pltpu.sync_copy(buf_v.at[1], sums_sp.at[pl.ds(sid * _CPW, _ROWS)])

    tgt_cp.wait()

    @plsc.parallel_loop(0, _B // _L, unroll=2)
    def _prep(g):
        tvec = tgt_v[pl.ds(g * _L, _L)]
        dup, last = plsc.scan_count(tvec)
        plsc.addupdate_scatter(
            cnt_v, [tvec], dup.astype(jnp.float32), mask=last)

    @plsc.parallel_loop(0, _SPW // _L, unroll=2)
    def _prep_idx(g):
        c = g // (_ROWS // _L)
        off = (g % (_ROWS // _L)) * _L
        idx_v[c, pl.ds(off, _L)] = tgt_v[pl.ds(s0 + g * _L, _L)]

    plsc.subcore_barrier()

    for c in range(_NCH):
        _wait_inbound(c)
        _scatter(c).start(add=True)
        if c >= 1:
            _scatter(c - 1).wait()
        if c + 1 < _NCH:
            _start_inbound(c + 1)
    _scatter(_NCH - 1).wait()

    plsc.subcore_barrier()

    c0 = sid * _CPW
    cs = jnp.minimum(c0, _C - _CPW)
    pltpu.sync_copy(sums_sp.at[pl.ds(cs, _CPW)], buf_v.at[0])
    cent_cp = pltpu.make_async_copy(
        cent_hbm.at[pl.ds(cs, _CPW)], buf_v.at[1], csem)
    cent_cp.start()

    @plsc.parallel_loop(0, _CT // _L, unroll=4)
    def _inv_cnt(i):
        inv_v[pl.ds(i * _L, _L)] = 1.0 / jnp.maximum(cnt_v[pl.ds(i * _L, _L)], 1.0)

    cent_cp.wait()

    accs0 = (zeros, zeros, zeros, zeros)

    @plsc.parallel_loop(0, _CPW, carry=accs0)
    def _class_loop(ci, accs):
        cw = cnt_v[pl.ds(cs + ci, _L)]
        iw = inv_v[pl.ds(cs + ci, _L)]
        cb = _bcast_lane(cw, 0)
        inv = _bcast_lane(iw, 0)
        keep = jnp.full((_L,), 0, jnp.int32) + (cs + ci - c0) >= 0
        cb = jnp.where(keep, cb, 0.0)
        accs = list(accs)
        for f in range(_FV):
            d = buf_v[0, ci, pl.ds(f * _L, _L)] * inv \
                - buf_v[1, ci, pl.ds(f * _L, _L)]
            a = jnp.abs(d)
            h = jnp.where(a < 1.0, 0.5 * d * d, a - 0.5)
            accs[f % 4] = accs[f % 4] + cb * h
        return tuple(accs)

    accs = _class_loop
    res_v[...] = (accs[0] + accs[1]) + (accs[2] + accs[3])
    pltpu.sync_copy(res_v, out_hbm.at[wid])


_sc_kernel = functools.partial(
    pl.kernel,
    out_type=jax.ShapeDtypeStruct((_NW, _L), jnp.float32),
    mesh=plsc.VectorSubcoreMesh(core_axis_name="c", subcore_axis_name="s"),
    compiler_params=pltpu.CompilerParams(
        use_tc_tiling_on_sc=False, needs_layout_passes=False),
    scratch_types=[
        pltpu.VMEM((_B,), jnp.int32),
        pltpu.VMEM((_NCH, _ROWS), jnp.int32),
        pltpu.VMEM((2, _ROWS, _D), jnp.float32),
        pltpu.VMEM((_CT,), jnp.float32),
        pltpu.VMEM((_CT,), jnp.float32),
        pltpu.VMEM((_L,), jnp.float32),
        pltpu.VMEM_SHARED((_CT, _D), jnp.float32),
        pltpu.SemaphoreType.DMA,
        pltpu.SemaphoreType.DMA,
        pltpu.SemaphoreType.DMA((2,)),
        pltpu.SemaphoreType.DMA((2,)),
    ],
)(_sc_body)


def _tc_reduce_body(x_ref, o_ref):
    o_ref[...] = jnp.sum(x_ref[...]).reshape(1, 1) * (1.0 / (_B * _D))


def kernel(modal1_inputs, modal2_inputs, targets, centers_param):
    partials = _sc_kernel(modal1_inputs, modal2_inputs, targets, centers_param)
    return jnp.sum(partials) * (1.0 / (_B * _D))
